# baseline (speedup 1.0000x reference)
import functools

import jax
import jax.numpy as jnp
from jax import lax
from jax.experimental import pallas as pl
from jax.experimental.pallas import tpu as pltpu
from jax.experimental.pallas import tpu_sc as plsc

T = 2048
H = 1024
INTER = 2048
I2 = 2 * INTER
E = 8
K = 2
S = T * K
TILE = 256
NT = (S + E * TILE) // TILE
P = NT * TILE


def _routing_metadata(top_k_index, top_k_weights):
    flat_e = top_k_index.reshape(-1).astype(jnp.int32)
    flat_w = top_k_weights.reshape(-1).astype(jnp.float32)
    onehot = (flat_e[:, None] == jnp.arange(E, dtype=jnp.int32)[None, :]).astype(jnp.int32)
    occ = jnp.cumsum(onehot, axis=0)
    counts = occ[-1]
    rank = jnp.take_along_axis(occ, flat_e[:, None], axis=1)[:, 0] - 1
    padded = ((counts + TILE - 1) // TILE) * TILE
    gend = jnp.cumsum(padded)
    gstart = gend - padded
    pos = gstart[flat_e] + rank
    n_active = (gend[-1] // TILE).astype(jnp.int32)
    w_pad = jnp.zeros((P,), jnp.float32).at[pos].set(flat_w)
    tile_base = jnp.arange(NT, dtype=jnp.int32) * TILE
    tile_expert = jnp.minimum(
        jnp.searchsorted(gend, tile_base, side="right").astype(jnp.int32), E - 1)
    meta = jnp.concatenate([tile_expert, n_active[None]])
    return w_pad, pos, meta


def _sc_mesh():
    info = plsc.get_sparse_core_info()
    return (plsc.VectorSubcoreMesh(core_axis_name="c", subcore_axis_name="s"),
            info.num_cores, info.num_subcores)


def _dispatch_rows(hidden, pos):
    mesh, nc, ns = _sc_mesh()
    nw = nc * ns
    tok_pw = T // nw
    pos3 = jnp.stack(
        [pos[0::2].reshape(nw, tok_pw), pos[1::2].reshape(nw, tok_pw)], axis=1)

    @functools.partial(
        pl.kernel,
        mesh=mesh,
        out_type=jax.ShapeDtypeStruct((P, H), jnp.float32),
        scratch_types=[
            pltpu.VMEM((2, tok_pw), jnp.int32),
            pltpu.VMEM((tok_pw, H), jnp.float32),
            pltpu.SemaphoreType.DMA,
            pltpu.SemaphoreType.DMA,
        ],
    )
    def dispatch_k(hid_hbm, pos_hbm, out_hbm, idx_v, rows_v, sem0, sem1):
        wid = lax.axis_index("s") * nc + lax.axis_index("c")
        pltpu.sync_copy(pos_hbm.at[wid], idx_v)
        pltpu.sync_copy(hid_hbm.at[pl.ds(wid * tok_pw, tok_pw)], rows_v)
        s0 = pltpu.async_copy(rows_v, out_hbm.at[idx_v.at[0]], sem0)
        s1 = pltpu.async_copy(rows_v, out_hbm.at[idx_v.at[1]], sem1)
        s0.wait()
        s1.wait()

    return dispatch_k(hidden, pos3)


NC2 = 4
CW = INTER // NC2


def _grouped_mlp(xg, gate_up_proj, down_proj, w_pad, meta):
    w2d = jnp.broadcast_to(w_pad[:, None], (P, 128))

    def body(meta_ref, xg_ref, gug_ref, guu_ref, dn_ref, w_ref, y_ref):
        t = pl.program_id(0)
        c = pl.program_id(1)

        @pl.when(t < meta_ref[NT])
        def _():
            x = xg_ref[...]
            gate = lax.dot_general(
                x, gug_ref[0], (((1,), (1,)), ((), ())),
                preferred_element_type=jnp.float32)
            up = lax.dot_general(
                x, guu_ref[0], (((1,), (1,)), ((), ())),
                preferred_element_type=jnp.float32)
            h = jax.nn.gelu(gate, approximate=True) * up
            ypart = lax.dot_general(
                h, dn_ref[0], (((1,), (1,)), ((), ())),
                preferred_element_type=jnp.float32)

            @pl.when(c == 0)
            def _():
                y_ref[...] = ypart

            @pl.when(c > 0)
            def _():
                acc = y_ref[...] + ypart
                y_ref[...] = jnp.where(c == NC2 - 1, acc * w_ref[:, :1], acc)

    grid_spec = pltpu.PrefetchScalarGridSpec(
        num_scalar_prefetch=1,
        grid=(NT, NC2),
        in_specs=[
            pl.BlockSpec((TILE, H), lambda t, c, m: (t, 0)),
            pl.BlockSpec((1, CW, H), lambda t, c, m: (m[t], c, 0)),
            pl.BlockSpec((1, CW, H), lambda t, c, m: (m[t], NC2 + c, 0)),
            pl.BlockSpec((1, H, CW), lambda t, c, m: (m[t], 0, c)),
            pl.BlockSpec((TILE, 128), lambda t, c, m: (t, 0)),
        ],
        out_specs=pl.BlockSpec((TILE, H), lambda t, c, m: (t, 0)),
    )
    return pl.pallas_call(
        body,
        grid_spec=grid_spec,
        out_shape=jax.ShapeDtypeStruct((P, H), jnp.float32),
        compiler_params=pltpu.CompilerParams(
            dimension_semantics=("arbitrary", "arbitrary")),
    )(meta, xg, gate_up_proj, gate_up_proj, down_proj, w2d)


def _combine(y, pos):
    mesh, nc, ns = _sc_mesh()
    nw = nc * ns
    tok_pw = T // nw
    ch = min(tok_pw, 32)
    n_ch = tok_pw // ch

    @functools.partial(
        pl.kernel,
        mesh=mesh,
        out_type=jax.ShapeDtypeStruct((T, H), jnp.float32),
        scratch_types=[
            pltpu.VMEM((2 * ch,), jnp.int32),
            pltpu.VMEM((2 * ch, H), jnp.float32),
            pltpu.VMEM((ch, H), jnp.float32),
            pltpu.SemaphoreType.DMA,
        ],
    )
    def combine_k(y_hbm, pos_hbm, out_hbm, idx_v, rows_v, acc_v, sem):
        wid = lax.axis_index("s") * nc + lax.axis_index("c")
        for c in range(n_ch):
            tbase = wid * tok_pw + c * ch
            pltpu.sync_copy(pos_hbm.at[pl.ds(2 * tbase, 2 * ch)], idx_v)
            pltpu.async_copy(y_hbm.at[idx_v], rows_v, sem).wait()

            def col_body(ci, j):
                a = rows_v[2 * j, pl.ds(ci * 16, 16)]
                b = rows_v[2 * j + 1, pl.ds(ci * 16, 16)]
                acc_v[j, pl.ds(ci * 16, 16)] = a + b
                return j

            def tok_body(j, _):
                lax.fori_loop(0, H // 16, col_body, j)
                return 0

            lax.fori_loop(0, ch, tok_body, 0)
            pltpu.sync_copy(acc_v, out_hbm.at[pl.ds(tbase, ch)])

    return combine_k(y, pos)


def kernel(hidden_states, top_k_index, top_k_weights, gate_up_proj, down_proj):
    w_pad, pos, meta = _routing_metadata(top_k_index, top_k_weights)
    xg = _dispatch_rows(hidden_states, pos)
    y = _grouped_mlp(xg, gate_up_proj, down_proj, w_pad, meta)
    return _combine(y, pos)


# speedup vs baseline: 1.4184x; 1.4184x over previous
import functools

import jax
import jax.numpy as jnp
from jax import lax
from jax.experimental import pallas as pl
from jax.experimental.pallas import tpu as pltpu
from jax.experimental.pallas import tpu_sc as plsc

T = 2048
H = 1024
INTER = 2048
I2 = 2 * INTER
E = 8
K = 2
S = T * K
TILE = 256
NT = (S + E * TILE) // TILE
P = NT * TILE


def _routing_metadata(top_k_index, top_k_weights):
    flat_e = top_k_index.reshape(-1).astype(jnp.int32)
    flat_w = top_k_weights.reshape(-1).astype(jnp.float32)
    onehot = (flat_e[:, None] == jnp.arange(E, dtype=jnp.int32)[None, :]).astype(jnp.int32)
    occ = jnp.cumsum(onehot, axis=0)
    counts = occ[-1]
    rank = jnp.take_along_axis(occ, flat_e[:, None], axis=1)[:, 0] - 1
    padded = ((counts + TILE - 1) // TILE) * TILE
    gend = jnp.cumsum(padded)
    gstart = gend - padded
    pos = gstart[flat_e] + rank
    n_active = (gend[-1] // TILE).astype(jnp.int32)
    w_pad = jnp.zeros((P,), jnp.float32).at[pos].set(flat_w)
    tile_base = jnp.arange(NT, dtype=jnp.int32) * TILE
    tile_expert = jnp.minimum(
        jnp.searchsorted(gend, tile_base, side="right").astype(jnp.int32), E - 1)
    meta = jnp.concatenate([tile_expert, n_active[None]])
    return w_pad, pos, meta


def _sc_mesh():
    info = plsc.get_sparse_core_info()
    return (plsc.VectorSubcoreMesh(core_axis_name="c", subcore_axis_name="s"),
            info.num_cores, info.num_subcores)


def _dispatch_rows(hidden, pos):
    mesh, nc, ns = _sc_mesh()
    nw = nc * ns
    tok_pw = T // nw
    pos3 = jnp.stack(
        [pos[0::2].reshape(nw, tok_pw), pos[1::2].reshape(nw, tok_pw)], axis=1)

    @functools.partial(
        pl.kernel,
        mesh=mesh,
        out_type=jax.ShapeDtypeStruct((P, H), jnp.float32),
        scratch_types=[
            pltpu.VMEM((2, tok_pw), jnp.int32),
            pltpu.VMEM((tok_pw, H), jnp.float32),
            pltpu.SemaphoreType.DMA,
            pltpu.SemaphoreType.DMA,
        ],
    )
    def dispatch_k(hid_hbm, pos_hbm, out_hbm, idx_v, rows_v, sem0, sem1):
        wid = lax.axis_index("s") * nc + lax.axis_index("c")
        pltpu.sync_copy(pos_hbm.at[wid], idx_v)
        pltpu.sync_copy(hid_hbm.at[pl.ds(wid * tok_pw, tok_pw)], rows_v)
        s0 = pltpu.async_copy(rows_v, out_hbm.at[idx_v.at[0]], sem0)
        s1 = pltpu.async_copy(rows_v, out_hbm.at[idx_v.at[1]], sem1)
        s0.wait()
        s1.wait()

    return dispatch_k(hidden, pos3)


def _grouped_mlp(xg, gate_up_proj, down_proj, w_pad, meta):
    w2d = jnp.broadcast_to(w_pad[:, None], (P, 128))

    def body(meta_ref, xg_ref, gu_ref, dn_ref, w_ref, y_ref):
        t = pl.program_id(0)

        @pl.when(t < meta_ref[NT])
        def _():
            x = xg_ref[...]
            gu = gu_ref[0]
            proj = lax.dot_general(
                x, gu, (((1,), (1,)), ((), ())),
                preferred_element_type=jnp.float32)
            gate = proj[:, :INTER]
            up = proj[:, INTER:]
            h = jax.nn.gelu(gate, approximate=True) * up
            dn = dn_ref[0]
            y = lax.dot_general(
                h, dn, (((1,), (1,)), ((), ())),
                preferred_element_type=jnp.float32)
            y_ref[...] = y * w_ref[:, :1]

    grid_spec = pltpu.PrefetchScalarGridSpec(
        num_scalar_prefetch=1,
        grid=(NT,),
        in_specs=[
            pl.BlockSpec((TILE, H), lambda t, m: (t, 0)),
            pl.BlockSpec((1, I2, H), lambda t, m: (m[t], 0, 0)),
            pl.BlockSpec((1, H, INTER), lambda t, m: (m[t], 0, 0)),
            pl.BlockSpec((TILE, 128), lambda t, m: (t, 0)),
        ],
        out_specs=pl.BlockSpec((TILE, H), lambda t, m: (t, 0)),
    )
    return pl.pallas_call(
        body,
        grid_spec=grid_spec,
        out_shape=jax.ShapeDtypeStruct((P, H), jnp.float32),
        compiler_params=pltpu.CompilerParams(
            dimension_semantics=("arbitrary",),
            vmem_limit_bytes=120 * 1024 * 1024),
    )(meta, xg, gate_up_proj, down_proj, w2d)


def _combine(y, pos):
    mesh, nc, ns = _sc_mesh()
    nw = nc * ns
    tok_pw = T // nw
    ch = min(tok_pw, 32)
    n_ch = tok_pw // ch

    @functools.partial(
        pl.kernel,
        mesh=mesh,
        out_type=jax.ShapeDtypeStruct((T, H), jnp.float32),
        scratch_types=[
            pltpu.VMEM((2 * ch,), jnp.int32),
            pltpu.VMEM((2 * ch, H), jnp.float32),
            pltpu.VMEM((ch, H), jnp.float32),
            pltpu.SemaphoreType.DMA,
        ],
    )
    def combine_k(y_hbm, pos_hbm, out_hbm, idx_v, rows_v, acc_v, sem):
        wid = lax.axis_index("s") * nc + lax.axis_index("c")
        for c in range(n_ch):
            tbase = wid * tok_pw + c * ch
            pltpu.sync_copy(pos_hbm.at[pl.ds(2 * tbase, 2 * ch)], idx_v)
            pltpu.async_copy(y_hbm.at[idx_v], rows_v, sem).wait()

            def col_body(ci, j):
                a = rows_v[2 * j, pl.ds(ci * 16, 16)]
                b = rows_v[2 * j + 1, pl.ds(ci * 16, 16)]
                acc_v[j, pl.ds(ci * 16, 16)] = a + b
                return j

            def tok_body(j, _):
                lax.fori_loop(0, H // 16, col_body, j)
                return 0

            lax.fori_loop(0, ch, tok_body, 0)
            pltpu.sync_copy(acc_v, out_hbm.at[pl.ds(tbase, ch)])

    return combine_k(y, pos)


def kernel(hidden_states, top_k_index, top_k_weights, gate_up_proj, down_proj):
    w_pad, pos, meta = _routing_metadata(top_k_index, top_k_weights)
    xg = _dispatch_rows(hidden_states, pos)
    y = _grouped_mlp(xg, gate_up_proj, down_proj, w_pad, meta)
    return _combine(y, pos)


# speedup vs baseline: 1.4489x; 1.0215x over previous
import functools

import jax
import jax.numpy as jnp
from jax import lax
from jax.experimental import pallas as pl
from jax.experimental.pallas import tpu as pltpu
from jax.experimental.pallas import tpu_sc as plsc

T = 2048
H = 1024
INTER = 2048
I2 = 2 * INTER
E = 8
K = 2
S = T * K
TILE = 256
NT = (S + E * TILE) // TILE
P = NT * TILE


def _routing_metadata(top_k_index, top_k_weights):
    flat_e = top_k_index.reshape(-1).astype(jnp.int32)
    flat_w = top_k_weights.reshape(-1).astype(jnp.float32)
    onehot = (flat_e[:, None] == jnp.arange(E, dtype=jnp.int32)[None, :]).astype(jnp.int32)
    occ = jnp.cumsum(onehot, axis=0)
    counts = occ[-1]
    rank = jnp.take_along_axis(occ, flat_e[:, None], axis=1)[:, 0] - 1
    padded = ((counts + TILE - 1) // TILE) * TILE
    gend = jnp.cumsum(padded)
    gstart = gend - padded
    pos = gstart[flat_e] + rank
    n_active = (gend[-1] // TILE).astype(jnp.int32)
    w_pad = jnp.zeros((P,), jnp.float32).at[pos].set(flat_w)
    tile_base = jnp.arange(NT, dtype=jnp.int32) * TILE
    tile_expert = jnp.minimum(
        jnp.searchsorted(gend, tile_base, side="right").astype(jnp.int32), E - 1)
    meta = jnp.concatenate([tile_expert, n_active[None]])
    return w_pad, pos, meta


def _sc_mesh():
    info = plsc.get_sparse_core_info()
    return (plsc.VectorSubcoreMesh(core_axis_name="c", subcore_axis_name="s"),
            info.num_cores, info.num_subcores)


def _dispatch_rows(hidden, pos):
    mesh, nc, ns = _sc_mesh()
    nw = nc * ns
    tok_pw = T // nw
    pos3 = jnp.stack(
        [pos[0::2].reshape(nw, tok_pw), pos[1::2].reshape(nw, tok_pw)], axis=1)

    @functools.partial(
        pl.kernel,
        mesh=mesh,
        out_type=jax.ShapeDtypeStruct((P, H), jnp.float32),
        scratch_types=[
            pltpu.VMEM((2, tok_pw), jnp.int32),
            pltpu.VMEM((tok_pw, H), jnp.float32),
            pltpu.SemaphoreType.DMA,
            pltpu.SemaphoreType.DMA,
        ],
    )
    def dispatch_k(hid_hbm, pos_hbm, out_hbm, idx_v, rows_v, sem0, sem1):
        wid = lax.axis_index("s") * nc + lax.axis_index("c")
        pltpu.sync_copy(pos_hbm.at[wid], idx_v)
        pltpu.sync_copy(hid_hbm.at[pl.ds(wid * tok_pw, tok_pw)], rows_v)
        s0 = pltpu.async_copy(rows_v, out_hbm.at[idx_v.at[0]], sem0)
        s1 = pltpu.async_copy(rows_v, out_hbm.at[idx_v.at[1]], sem1)
        s0.wait()
        s1.wait()

    return dispatch_k(hidden, pos3)


def _grouped_mlp(xg, gate_up_proj, down_proj, w_pad, meta):
    w2d = jnp.broadcast_to(w_pad[:, None], (P, 128))

    def body(meta_ref, xg_ref, gu_ref, dn_ref, w_ref, y_ref):
        t = pl.program_id(0)

        @pl.when(t < meta_ref[NT])
        def _():
            x = xg_ref[...]
            gu = gu_ref[0]
            proj = lax.dot_general(
                x, gu, (((1,), (1,)), ((), ())),
                preferred_element_type=jnp.float32)
            gate = proj[:, :INTER]
            up = proj[:, INTER:]
            h = jax.nn.gelu(gate, approximate=True) * up
            dn = dn_ref[0]
            y = lax.dot_general(
                h, dn, (((1,), (1,)), ((), ())),
                preferred_element_type=jnp.float32)
            y_ref[...] = y * w_ref[:, :1]

    grid_spec = pltpu.PrefetchScalarGridSpec(
        num_scalar_prefetch=1,
        grid=(NT,),
        in_specs=[
            pl.BlockSpec((TILE, H), lambda t, m: (t, 0)),
            pl.BlockSpec((1, I2, H), lambda t, m: (m[t], 0, 0)),
            pl.BlockSpec((1, H, INTER), lambda t, m: (m[t], 0, 0)),
            pl.BlockSpec((TILE, 128), lambda t, m: (t, 0)),
        ],
        out_specs=pl.BlockSpec((TILE, H), lambda t, m: (t, 0)),
    )
    return pl.pallas_call(
        body,
        grid_spec=grid_spec,
        out_shape=jax.ShapeDtypeStruct((P, H), jnp.float32),
        compiler_params=pltpu.CompilerParams(
            dimension_semantics=("arbitrary",),
            vmem_limit_bytes=120 * 1024 * 1024),
    )(meta, xg, gate_up_proj, down_proj, w2d)


def _combine(y, pos):
    mesh, nc, ns = _sc_mesh()
    nw = nc * ns
    tok_pw = T // nw
    n_ch = 4
    ch = tok_pw // n_ch

    @functools.partial(
        pl.kernel,
        mesh=mesh,
        out_type=jax.ShapeDtypeStruct((T, H), jnp.float32),
        scratch_types=[
            pltpu.VMEM((n_ch, 2 * ch), jnp.int32),
            pltpu.VMEM((2 * ch, H), jnp.float32),
            pltpu.VMEM((2 * ch, H), jnp.float32),
            pltpu.VMEM((ch, H), jnp.float32),
            pltpu.VMEM((ch, H), jnp.float32),
            pltpu.SemaphoreType.DMA,
            pltpu.SemaphoreType.DMA,
            pltpu.SemaphoreType.DMA,
            pltpu.SemaphoreType.DMA,
        ],
    )
    def combine_k(y_hbm, pos_hbm, out_hbm, idx_v, rows0, rows1, acc0, acc1,
                  gs0, gs1, ws0, ws1):
        wid = lax.axis_index("s") * nc + lax.axis_index("c")
        pltpu.sync_copy(pos_hbm.at[wid], idx_v)
        rows = (rows0, rows1)
        accs = (acc0, acc1)
        gsems = (gs0, gs1)
        wsems = (ws0, ws1)
        gath = [pltpu.async_copy(y_hbm.at[idx_v.at[0]], rows0, gs0),
                pltpu.async_copy(y_hbm.at[idx_v.at[1]], rows1, gs1)]
        wrs = [None, None]
        for c in range(n_ch):
            b = c % 2
            gath[b].wait()
            if c >= 2:
                wrs[b].wait()

            def col_body(ci, j, _b=b):
                a = rows[_b][2 * j, pl.ds(ci * 16, 16)]
                bb = rows[_b][2 * j + 1, pl.ds(ci * 16, 16)]
                accs[_b][j, pl.ds(ci * 16, 16)] = a + bb
                return j

            def tok_body(j, _, _cb=col_body):
                lax.fori_loop(0, H // 16, _cb, j)
                return 0

            lax.fori_loop(0, ch, tok_body, 0)
            wrs[b] = pltpu.async_copy(
                accs[b], out_hbm.at[pl.ds(wid * tok_pw + c * ch, ch)],
                wsems[b])
            if c + 2 < n_ch:
                gath[b] = pltpu.async_copy(
                    y_hbm.at[idx_v.at[c + 2]], rows[b], gsems[b])
        wrs[0].wait()
        wrs[1].wait()

    return combine_k(y, pos.reshape(nw, n_ch, 2 * ch))


def kernel(hidden_states, top_k_index, top_k_weights, gate_up_proj, down_proj):
    w_pad, pos, meta = _routing_metadata(top_k_index, top_k_weights)
    xg = _dispatch_rows(hidden_states, pos)
    y = _grouped_mlp(xg, gate_up_proj, down_proj, w_pad, meta)
    return _combine(y, pos)
